# baseline (speedup 1.0000x reference)
import functools

import jax
import jax.numpy as jnp
from jax import lax
from jax.experimental import pallas as pl
from jax.experimental.pallas import tpu as pltpu
from jax.experimental.pallas import tpu_sc as plsc

N = 10000
E = 320000
D_IN = 128
NC, NS, NW = 2, 16, 32
B = 128
K = 80
EPAD = NW * K * B
NPAD = 10240
PADROW = N
RPS = NPAD // NS

_mesh = plsc.VectorSubcoreMesh(core_axis_name="c", subcore_axis_name="s")
_sc_params = pltpu.CompilerParams(use_tc_tiling_on_sc=False)


@functools.partial(
    pl.kernel,
    out_type=jax.ShapeDtypeStruct((NC, NPAD, 16), jnp.float32),
    mesh=_mesh,
    compiler_params=_sc_params,
    scratch_types=[
        pltpu.VMEM((K // 4, 4 * B), jnp.int32),
        pltpu.VMEM((4 * B, 16), jnp.float32),
        pltpu.VMEM_SHARED((NPAD, 16), jnp.float32),
        pltpu.SemaphoreType.DMA,
    ],
)
def _sc_degree(dst_hbm, zeros_hbm, ones_hbm, out_hbm, idx_v, ones_v, acc, sem):
    cid = lax.axis_index("c")
    sid = lax.axis_index("s")
    wid = sid * NC + cid
    sl = pl.ds(sid * RPS, RPS)
    pltpu.sync_copy(zeros_hbm.at[sl], acc.at[sl])
    pltpu.sync_copy(ones_hbm, ones_v)
    pltpu.sync_copy(dst_hbm.at[wid], idx_v)
    plsc.subcore_barrier()

    @pl.loop(0, K // 4)
    def _(j):
        pltpu.async_copy(ones_v, acc.at[idx_v.at[j]], sem, add=True)

    @pl.loop(0, K // 4)
    def _(j):
        pltpu.make_async_copy(ones_v, acc.at[idx_v.at[j]], sem).wait()

    plsc.subcore_barrier()
    pltpu.sync_copy(acc.at[sl], out_hbm.at[cid].at[sl])


def _make_sc_scatter(H):
    @functools.partial(
        pl.kernel,
        out_type=jax.ShapeDtypeStruct((NC, NPAD, H), jnp.float32),
        mesh=_mesh,
        compiler_params=_sc_params,
        scratch_types=[
            pltpu.VMEM((K // 4, 4 * B), jnp.int32),
            pltpu.VMEM((K // 4, 4 * B), jnp.int32),
            pltpu.VMEM((2, 4 * B, H), jnp.float32),
            pltpu.VMEM_SHARED((NPAD, H), jnp.float32),
            pltpu.VMEM_SHARED((NPAD, H), jnp.float32),
            pltpu.SemaphoreType.DMA,
            pltpu.SemaphoreType.DMA,
            pltpu.SemaphoreType.DMA,
        ],
    )
    def scat(g_hbm, src_hbm, dst_hbm, zeros_hbm, out_hbm,
             src_v, dst_v, rows_v, acc, gsh, gsemA, gsemB, ssem):
        cid = lax.axis_index("c")
        sid = lax.axis_index("s")
        wid = sid * NC + cid
        sl = pl.ds(sid * RPS, RPS)
        pltpu.sync_copy(zeros_hbm.at[sl], acc.at[sl])
        pltpu.sync_copy(g_hbm.at[sl], gsh.at[sl])
        pltpu.sync_copy(src_hbm.at[wid], src_v)
        pltpu.sync_copy(dst_hbm.at[wid], dst_v)
        plsc.subcore_barrier()

        def fire_gather(slot, slab, sem):
            return pltpu.async_copy(gsh.at[src_v.at[slab]],
                                    rows_v.at[slot], sem)

        def fire_scatter(slot, slab):
            return pltpu.async_copy(rows_v.at[slot],
                                    acc.at[dst_v.at[slab]], ssem, add=True)

        NS4 = K // 4
        fire_gather(0, 0, gsemA)

        @pl.loop(0, NS4 // 2)
        def _(g):
            base = g * 2
            hB = fire_gather(1, base + 1, gsemB)
            pltpu.make_async_copy(gsh.at[src_v.at[base]],
                                  rows_v.at[0], gsemA).wait()
            fire_scatter(0, base).wait()

            @pl.when(g < NS4 // 2 - 1)
            def _():
                fire_gather(0, base + 2, gsemA)

            hB.wait()
            fire_scatter(1, base + 1).wait()

        plsc.subcore_barrier()
        pltpu.sync_copy(acc.at[sl], out_hbm.at[cid].at[sl])

    return scat


_sc_scatter32 = _make_sc_scatter(32)
_sc_scatter16 = _make_sc_scatter(16)


def _tc_call(body, out_shape):
    return pl.pallas_call(body, out_shape=out_shape)


def _row_mask(val, nrows):
    rows = lax.broadcasted_iota(jnp.int32, val.shape, 0)
    return jnp.where(rows < nrows, val, 0.0)


def _tc_h1_body(xg_ref, w_ref, o_ref):
    o_ref[...] = jnp.dot(xg_ref[...], w_ref[...],
                         preferred_element_type=jnp.float32)


def _tc_dis_body(deg_ref, d8_ref):
    d8_ref[...] = lax.rsqrt(deg_ref[0] + deg_ref[1] + 1.0)


def _tc_g1_body(h_ref, d4h_ref, dup_ref, g_ref):
    dis4 = jnp.dot(d4h_ref[...], dup_ref[...],
                   preferred_element_type=jnp.float32)
    g_ref[...] = h_ref[...] * dis4


def _tc_mid1_body(p_ref, g_ref, d4h_ref, dup_ref, b_ref, w_ref, o_ref):
    dis4 = jnp.dot(d4h_ref[...], dup_ref[...],
                   preferred_element_type=jnp.float32)
    s = p_ref[0] + p_ref[1] + g_ref[...]
    t = jnp.maximum(s * dis4 + b_ref[...], 0.0)
    m = jnp.dot(t, w_ref[...], preferred_element_type=jnp.float32)
    o_ref[...] = _row_mask(m * d4h_ref[...], N // 4)


def _tc_mid2_body(p_ref, g_ref, d8_ref, b_ref, w_ref, o_ref):
    s = p_ref[0] + p_ref[1] + g_ref[...]
    t = jnp.maximum(s * d8_ref[...] + b_ref[...], 0.0)
    m = jnp.dot(t, w_ref[...], preferred_element_type=jnp.float32)
    o_ref[...] = _row_mask(m * d8_ref[...], N // 8)


def _tc_out_body(p_ref, g_ref, d8_ref, b_ref, o_ref):
    s = p_ref[0] + p_ref[1] + g_ref[...]
    o_ref[...] = s * d8_ref[...] + b_ref[...]


@jax.jit
def kernel(x, edge_index, W1, b1, W2, b2, W3, b3):
    src = edge_index[0].astype(jnp.int32)
    dst = edge_index[1].astype(jnp.int32)
    pad = jnp.full((EPAD - E,), PADROW, jnp.int32)
    src3 = jnp.concatenate([src, pad]).reshape(NW, K // 4, 4 * B)
    dst3 = jnp.concatenate([dst, pad]).reshape(NW, K // 4, 4 * B)

    z16 = jnp.zeros((NPAD, 16), jnp.float32)
    z32 = jnp.zeros((NPAD, 32), jnp.float32)
    ones = jnp.ones((4 * B, 16), jnp.float32)
    W3p = jnp.pad(W3, ((0, 0), (0, 8)))
    eye4 = jnp.eye(4, dtype=jnp.float32)
    eye8 = jnp.eye(8, dtype=jnp.float32)
    W1b = jnp.kron(eye4, W1)
    W2b = jnp.kron(eye4, W2)
    W3b = jnp.kron(eye8, W3p)
    b1t = jnp.tile(b1, 4).reshape(1, 128)
    b2t = jnp.tile(b2, 8).reshape(1, 128)
    b3t = jnp.tile(jnp.pad(b3, (0, 8)), 8).reshape(1, 128)
    xg = jnp.pad(x, ((0, NPAD - N), (0, 0))).reshape(NPAD // 4, 512)

    f32 = jnp.float32
    degp = _sc_degree(dst3, z16, ones).reshape(2, NPAD // 8, 128)
    h1p = _tc_call(_tc_h1_body,
                   jax.ShapeDtypeStruct((NPAD // 4, 128), f32))(xg, W1b)
    dis8 = _tc_call(_tc_dis_body,
                    jax.ShapeDtypeStruct((NPAD // 8, 128), f32))(degp)
    dis4h = dis8.reshape(NPAD // 4, 64)
    eye16 = jnp.eye(16, dtype=jnp.float32)
    dup = jnp.kron(eye4, jnp.concatenate([eye16, eye16], axis=1))

    g1p = _tc_call(_tc_g1_body,
                   jax.ShapeDtypeStruct((NPAD // 4, 128), f32))(
                       h1p, dis4h, dup)
    p1 = _sc_scatter32(g1p.reshape(NPAD, 32), src3, dst3, z32)
    g2h = _tc_call(_tc_mid1_body,
                   jax.ShapeDtypeStruct((NPAD // 4, 64), f32))(
                       p1.reshape(2, NPAD // 4, 128), g1p, dis4h, dup,
                       b1t, W2b)
    p2 = _sc_scatter16(g2h.reshape(NPAD, 16), src3, dst3, z16)
    g2p = g2h.reshape(NPAD // 8, 128)
    g3p = _tc_call(_tc_mid2_body,
                   jax.ShapeDtypeStruct((NPAD // 8, 128), f32))(
                       p2.reshape(2, NPAD // 8, 128), g2p, dis8, b2t, W3b)
    p3 = _sc_scatter16(g3p.reshape(NPAD, 16), src3, dst3, z16)
    vp = _tc_call(_tc_out_body,
                  jax.ShapeDtypeStruct((NPAD // 8, 128), f32))(
                      p3.reshape(2, NPAD // 8, 128), g3p, dis8, b3t)
    return vp.reshape(NPAD, 16)[0:N, 0:8]


# speedup vs baseline: 1.1999x; 1.1999x over previous
import functools

import jax
import jax.numpy as jnp
from jax import lax
from jax.experimental import pallas as pl
from jax.experimental.pallas import tpu as pltpu
from jax.experimental.pallas import tpu_sc as plsc

N = 10000
E = 320000
D_IN = 128
NC, NS, NW = 2, 16, 32
B = 128
K = 80
EPAD = NW * K * B
NPAD = 10240
PADROW = N
RPS = NPAD // NS

_mesh = plsc.VectorSubcoreMesh(core_axis_name="c", subcore_axis_name="s")
_sc_params = pltpu.CompilerParams(use_tc_tiling_on_sc=False)


def _zero_acc(zbuf, acc, sid, H, sem):
    @pl.loop(0, 40)
    def _(r):
        @pl.loop(0, H // 16)
        def _(c):
            zbuf[r, pl.ds(c * 16, 16)] = jnp.zeros((16,), jnp.float32)

    @pl.loop(0, RPS // 40)
    def _(i):
        pltpu.async_copy(zbuf, acc.at[pl.ds(sid * RPS + i * 40, 40)], sem)

    @pl.loop(0, RPS // 40)
    def _(i):
        pltpu.make_async_copy(
            zbuf, acc.at[pl.ds(sid * RPS + i * 40, 40)], sem).wait()


NSLAB = E // 512
WSLAB = 20


@functools.partial(
    pl.kernel,
    out_type=jax.ShapeDtypeStruct((NC, NPAD, 16), jnp.float32),
    mesh=_mesh,
    compiler_params=_sc_params,
    scratch_types=[
        pltpu.VMEM((WSLAB, 4 * B), jnp.int32),
        pltpu.VMEM((4 * B, 16), jnp.float32),
        pltpu.VMEM((40, 16), jnp.float32),
        pltpu.VMEM_SHARED((NPAD, 16), jnp.float32),
        pltpu.SemaphoreType.DMA,
    ],
)
def _sc_degree(edge_hbm, ones_hbm, out_hbm, idx_v, ones_v, zbuf, acc, sem):
    cid = lax.axis_index("c")
    sid = lax.axis_index("s")
    wid = sid * NC + cid
    sl = pl.ds(sid * RPS, RPS)
    start = wid * WSLAB
    _zero_acc(zbuf, acc, sid, 16, sem)
    pltpu.sync_copy(ones_hbm, ones_v)

    @pl.when(wid < NW - 1)
    def _():
        pltpu.sync_copy(edge_hbm.at[1].at[pl.ds(start, WSLAB)], idx_v)

    @pl.when(wid == NW - 1)
    def _():
        pltpu.sync_copy(edge_hbm.at[1].at[pl.ds(start, 5)],
                        idx_v.at[pl.ds(0, 5)])

    plsc.subcore_barrier()

    def deg_slabs(nslab):
        @pl.loop(0, nslab)
        def _(j):
            pltpu.async_copy(ones_v, acc.at[idx_v.at[j]], sem, add=True)

        @pl.loop(0, nslab)
        def _(j):
            pltpu.make_async_copy(ones_v, acc.at[idx_v.at[j]], sem).wait()

    @pl.when(wid < NW - 1)
    def _():
        deg_slabs(WSLAB)

    @pl.when(wid == NW - 1)
    def _():
        deg_slabs(5)

    plsc.subcore_barrier()
    pltpu.sync_copy(acc.at[sl], out_hbm.at[cid].at[sl])


def _make_sc_scatter(H):
    @functools.partial(
        pl.kernel,
        out_type=jax.ShapeDtypeStruct((NC, NPAD, H), jnp.float32),
        mesh=_mesh,
        compiler_params=_sc_params,
        scratch_types=[
            pltpu.VMEM((WSLAB, 4 * B), jnp.int32),
            pltpu.VMEM((WSLAB, 4 * B), jnp.int32),
            pltpu.VMEM((2, 4 * B, H), jnp.float32),
            pltpu.VMEM((40, H), jnp.float32),
            pltpu.VMEM_SHARED((NPAD, H), jnp.float32),
            pltpu.VMEM_SHARED((NPAD, H), jnp.float32),
            pltpu.SemaphoreType.DMA,
            pltpu.SemaphoreType.DMA,
            pltpu.SemaphoreType.DMA,
        ],
    )
    def scat(g_hbm, edge_hbm, out_hbm,
             src_v, dst_v, rows_v, zbuf, acc, gsh, gsemA, gsemB, ssem):
        cid = lax.axis_index("c")
        sid = lax.axis_index("s")
        wid = sid * NC + cid
        sl = pl.ds(sid * RPS, RPS)
        start = wid * WSLAB
        pltpu.sync_copy(g_hbm.at[sl], gsh.at[sl])
        _zero_acc(zbuf, acc, sid, H, gsemA)

        @pl.when(wid < NW - 1)
        def _():
            pltpu.sync_copy(edge_hbm.at[0].at[pl.ds(start, WSLAB)], src_v)
            pltpu.sync_copy(edge_hbm.at[1].at[pl.ds(start, WSLAB)], dst_v)

        @pl.when(wid == NW - 1)
        def _():
            pltpu.sync_copy(edge_hbm.at[0].at[pl.ds(start, 5)],
                            src_v.at[pl.ds(0, 5)])
            pltpu.sync_copy(edge_hbm.at[1].at[pl.ds(start, 5)],
                            dst_v.at[pl.ds(0, 5)])

        plsc.subcore_barrier()

        def fire_gather(slot, slab, sem):
            return pltpu.async_copy(gsh.at[src_v.at[slab]],
                                    rows_v.at[slot], sem)

        def fire_scatter(slot, slab):
            return pltpu.async_copy(rows_v.at[slot],
                                    acc.at[dst_v.at[slab]], ssem, add=True)

        @pl.when(wid < NW - 1)
        def _():
            fire_gather(0, 0, gsemA)

            @pl.loop(0, WSLAB // 2)
            def _(g):
                base = g * 2
                hB = fire_gather(1, base + 1, gsemB)
                pltpu.make_async_copy(gsh.at[src_v.at[base]],
                                      rows_v.at[0], gsemA).wait()
                fire_scatter(0, base).wait()

                @pl.when(g < WSLAB // 2 - 1)
                def _():
                    fire_gather(0, base + 2, gsemA)

                hB.wait()
                fire_scatter(1, base + 1).wait()

        @pl.when(wid == NW - 1)
        def _():
            @pl.loop(0, 5)
            def _(j):
                pltpu.sync_copy(gsh.at[src_v.at[j]], rows_v.at[0])
                pltpu.sync_copy(rows_v.at[0], acc.at[dst_v.at[j]], add=True)

        plsc.subcore_barrier()
        pltpu.sync_copy(acc.at[sl], out_hbm.at[cid].at[sl])

    return scat


_sc_scatter32 = _make_sc_scatter(32)
_sc_scatter16 = _make_sc_scatter(16)


def _tc_call(body, out_shape):
    return pl.pallas_call(body, out_shape=out_shape)


def _row_mask(val, nrows):
    rows = lax.broadcasted_iota(jnp.int32, val.shape, 0)
    return jnp.where(rows < nrows, val, 0.0)


def _tc_h1_body(xg_ref, w_ref, o_ref):
    o_ref[...] = jnp.dot(xg_ref[...], w_ref[...],
                         preferred_element_type=jnp.float32)


def _tc_dis_body(deg_ref, d8_ref):
    d8_ref[...] = lax.rsqrt(deg_ref[0] + deg_ref[1] + 1.0)


def _tc_g1_body(h_ref, d4h_ref, dup_ref, g_ref):
    dis4 = jnp.dot(d4h_ref[...], dup_ref[...],
                   preferred_element_type=jnp.float32)
    g_ref[...] = h_ref[...] * dis4


def _tc_mid1_body(p_ref, g_ref, d4h_ref, dup_ref, b_ref, w_ref, o_ref):
    dis4 = jnp.dot(d4h_ref[...], dup_ref[...],
                   preferred_element_type=jnp.float32)
    s = p_ref[0] + p_ref[1] + g_ref[...]
    t = jnp.maximum(s * dis4 + b_ref[...], 0.0)
    m = jnp.dot(t, w_ref[...], preferred_element_type=jnp.float32)
    o_ref[...] = _row_mask(m * d4h_ref[...], N // 4)


def _tc_mid2_body(p_ref, g_ref, d8_ref, b_ref, w_ref, o_ref):
    s = p_ref[0] + p_ref[1] + g_ref[...]
    t = jnp.maximum(s * d8_ref[...] + b_ref[...], 0.0)
    m = jnp.dot(t, w_ref[...], preferred_element_type=jnp.float32)
    o_ref[...] = _row_mask(m * d8_ref[...], N // 8)


def _tc_out_body(p_ref, g_ref, d8_ref, b_ref, o_ref):
    s = p_ref[0] + p_ref[1] + g_ref[...]
    o_ref[...] = s * d8_ref[...] + b_ref[...]


@jax.jit
def kernel(x, edge_index, W1, b1, W2, b2, W3, b3):
    edge2 = edge_index.astype(jnp.int32).reshape(2, NSLAB, 4 * B)

    ones = jnp.ones((4 * B, 16), jnp.float32)
    W3p = jnp.pad(W3, ((0, 0), (0, 8)))
    eye4 = jnp.eye(4, dtype=jnp.float32)
    eye8 = jnp.eye(8, dtype=jnp.float32)
    W1b = jnp.kron(eye4, W1)
    W2b = jnp.kron(eye4, W2)
    W3b = jnp.kron(eye8, W3p)
    b1t = jnp.tile(b1, 4).reshape(1, 128)
    b2t = jnp.tile(b2, 8).reshape(1, 128)
    b3t = jnp.tile(jnp.pad(b3, (0, 8)), 8).reshape(1, 128)
    xg = jnp.pad(x, ((0, NPAD - N), (0, 0))).reshape(NPAD // 4, 512)

    f32 = jnp.float32
    degp = _sc_degree(edge2, ones).reshape(2, NPAD // 8, 128)
    h1p = _tc_call(_tc_h1_body,
                   jax.ShapeDtypeStruct((NPAD // 4, 128), f32))(xg, W1b)
    dis8 = _tc_call(_tc_dis_body,
                    jax.ShapeDtypeStruct((NPAD // 8, 128), f32))(degp)
    dis4h = dis8.reshape(NPAD // 4, 64)
    eye16 = jnp.eye(16, dtype=jnp.float32)
    dup = jnp.kron(eye4, jnp.concatenate([eye16, eye16], axis=1))

    g1p = _tc_call(_tc_g1_body,
                   jax.ShapeDtypeStruct((NPAD // 4, 128), f32))(
                       h1p, dis4h, dup)
    p1 = _sc_scatter32(g1p.reshape(NPAD, 32), edge2)
    g2h = _tc_call(_tc_mid1_body,
                   jax.ShapeDtypeStruct((NPAD // 4, 64), f32))(
                       p1.reshape(2, NPAD // 4, 128), g1p, dis4h, dup,
                       b1t, W2b)
    p2 = _sc_scatter16(g2h.reshape(NPAD, 16), edge2)
    g2p = g2h.reshape(NPAD // 8, 128)
    g3p = _tc_call(_tc_mid2_body,
                   jax.ShapeDtypeStruct((NPAD // 8, 128), f32))(
                       p2.reshape(2, NPAD // 8, 128), g2p, dis8, b2t, W3b)
    p3 = _sc_scatter16(g3p.reshape(NPAD, 16), edge2)
    vp = _tc_call(_tc_out_body,
                  jax.ShapeDtypeStruct((NPAD // 8, 128), f32))(
                      p3.reshape(2, NPAD // 8, 128), g3p, dis8, b3t)
    return vp.reshape(NPAD, 16)[0:N, 0:8]
